# baseline (speedup 1.0000x reference)
import functools

import jax
import jax.numpy as jnp
from jax import lax
from jax.experimental import pallas as pl
from jax.experimental.pallas import tpu as pltpu
from jax.experimental.pallas import tpu_sc as plsc

NC = 2
NS = 16
NW = NC * NS
CHUNK = 64


def _gather_rows(table, flat_idx, n, d):
    b_per_w = n // NW
    n_chunks = b_per_w // CHUNK
    mesh = plsc.VectorSubcoreMesh(
        core_axis_name="c", subcore_axis_name="s",
        num_cores=NC, num_subcores=NS)

    @functools.partial(
        pl.kernel,
        mesh=mesh,
        compiler_params=pltpu.CompilerParams(use_tc_tiling_on_sc=False),
        out_type=jax.ShapeDtypeStruct((n, d), jnp.float32),
        scratch_types=[
            pltpu.VMEM((b_per_w,), jnp.int32),
            pltpu.VMEM((CHUNK, d), jnp.float32),
            pltpu.VMEM((CHUNK, d), jnp.float32),
            pltpu.SemaphoreType.DMA,
            pltpu.SemaphoreType.DMA,
        ],
    )
    def run(table_hbm, idx_hbm, out_hbm, idx_v, buf0, buf1, sem0, sem1):
        wid = lax.axis_index("s") * NC + lax.axis_index("c")
        base = wid * b_per_w
        pltpu.sync_copy(idx_hbm.at[pl.ds(base, b_per_w)], idx_v)

        bufs = (buf0, buf1)
        sems = (sem0, sem1)

        def start_gather(c):
            return pltpu.async_copy(
                table_hbm.at[idx_v.at[pl.ds(c * CHUNK, CHUNK)]],
                bufs[c % 2], sems[c % 2])

        descs = [None] * n_chunks
        descs[0] = start_gather(0)
        for c in range(n_chunks):
            if c + 1 < n_chunks:
                descs[c + 1] = start_gather(c + 1)
            descs[c].wait()
            pltpu.sync_copy(bufs[c % 2],
                            out_hbm.at[pl.ds(base + c * CHUNK, CHUNK)])

    return run(table, flat_idx)


def kernel(idx, targets, token_embedding_table):
    del targets
    b, t = idx.shape
    n = b * t
    d = token_embedding_table.shape[1]
    flat_idx = idx.reshape(n).astype(jnp.int32)
    return _gather_rows(token_embedding_table, flat_idx, n, d)


# speedup vs baseline: 1.0428x; 1.0428x over previous
import functools

import jax
import jax.numpy as jnp
from jax import lax
from jax.experimental import pallas as pl
from jax.experimental.pallas import tpu as pltpu
from jax.experimental.pallas import tpu_sc as plsc

NC = 2
NS = 16
NW = NC * NS
CHUNK = 32


def _gather_rows(table, flat_idx, n, d):
    v = table.shape[0]
    b_per_w = n // NW
    n_pairs = b_per_w // (2 * CHUNK)
    mesh = plsc.VectorSubcoreMesh(
        core_axis_name="c", subcore_axis_name="s",
        num_cores=NC, num_subcores=NS)

    @functools.partial(
        pl.kernel,
        mesh=mesh,
        compiler_params=pltpu.CompilerParams(use_tc_tiling_on_sc=False),
        out_type=jax.ShapeDtypeStruct((n, d), jnp.float32),
        scratch_types=[
            pltpu.VMEM((b_per_w,), jnp.int32),
            pltpu.VMEM((CHUNK, d), jnp.float32),
            pltpu.VMEM((CHUNK, d), jnp.float32),
            pltpu.VMEM_SHARED((v, d), jnp.float32),
            pltpu.SemaphoreType.DMA,
            pltpu.SemaphoreType.DMA,
            pltpu.SemaphoreType.DMA,
            pltpu.SemaphoreType.DMA,
        ],
    )
    def run(table_hbm, idx_hbm, out_hbm, idx_v, buf0, buf1, table_sh,
            sg0, sg1, sw0, sw1):
        wid = lax.axis_index("s") * NC + lax.axis_index("c")
        sid = lax.axis_index("s")
        base = wid * b_per_w

        @pl.when(sid < 8)
        def _():
            pltpu.sync_copy(table_hbm.at[pl.ds(sid * 125, 125)],
                            table_sh.at[pl.ds(sid * 125, 125)])
        pltpu.sync_copy(idx_hbm.at[pl.ds(base, b_per_w)], idx_v)
        plsc.subcore_barrier()

        def start_gather(c, buf, sem):
            return pltpu.async_copy(
                table_sh.at[idx_v.at[pl.ds(c * CHUNK, CHUNK)]], buf, sem)

        def wait_gather(buf, sem):
            pltpu.make_async_copy(
                table_sh.at[idx_v.at[pl.ds(0, CHUNK)]], buf, sem).wait()

        def start_write(c, buf, sem):
            return pltpu.async_copy(
                buf, out_hbm.at[pl.ds(base + c * CHUNK, CHUNK)], sem)

        def wait_write(buf, sem):
            pltpu.make_async_copy(
                buf, out_hbm.at[pl.ds(base, CHUNK)], sem).wait()

        start_gather(0, buf0, sg0)
        start_gather(1, buf1, sg1)

        def body(g, carry):
            wait_gather(buf0, sg0)
            start_write(2 * g, buf0, sw0)
            wait_gather(buf1, sg1)
            start_write(2 * g + 1, buf1, sw1)

            @pl.when(g + 1 < n_pairs)
            def _():
                wait_write(buf0, sw0)
                start_gather(2 * g + 2, buf0, sg0)
                wait_write(buf1, sw1)
                start_gather(2 * g + 3, buf1, sg1)

            return carry

        lax.fori_loop(0, n_pairs, body, 0)
        wait_write(buf0, sw0)
        wait_write(buf1, sw1)

    return run(table, flat_idx)


def kernel(idx, targets, token_embedding_table):
    del targets
    b, t = idx.shape
    n = b * t
    d = token_embedding_table.shape[1]
    flat_idx = idx.reshape(n).astype(jnp.int32)
    return _gather_rows(token_embedding_table, flat_idx, n, d)
